# baseline (speedup 1.0000x reference)
import functools

import jax
import jax.numpy as jnp
from jax import lax
from jax.experimental import pallas as pl
from jax.experimental.pallas import tpu as pltpu
from jax.experimental.pallas import tpu_sc as plsc

OX, OY, OZ = 8, 8, 8
VOX = OX * OY * OZ
L = 16
NEG = -3.0e38


def _mask_tc(params3, x2, y2, z2, n_rois, P):
    CB = 128
    W16 = CB // L
    nrows = P // CB

    def body(prm_ref, x_ref, y_ref, z_ref, o_ref):
        cx = prm_ref[0, 0, 0]
        cy = prm_ref[0, 0, 1]
        cz = prm_ref[0, 0, 2]
        ca = prm_ref[0, 0, 3]
        sa = prm_ref[0, 0, 4]
        hdx = prm_ref[0, 0, 5]
        hdy = prm_ref[0, 0, 6]
        dz = prm_ref[0, 0, 7]
        px = x_ref[...] - cx
        py = y_ref[...] - cy
        pz = z_ref[...] - cz
        lx = px * ca - py * sa
        ly = px * sa + py * ca
        inb = ((jnp.abs(lx) < hdx) & (jnp.abs(ly) < hdy)
               & (pz > 0.0) & (pz < dz))
        bit = lax.broadcasted_iota(jnp.int32, (nrows, CB), 1) & (L - 1)
        pwf = (jnp.int32(1) << bit).astype(jnp.float32)
        m = inb.astype(jnp.float32) * pwf
        gi = lax.broadcasted_iota(jnp.int32, (CB, W16), 0) // L
        gj = lax.broadcasted_iota(jnp.int32, (CB, W16), 1)
        gm = (gi == gj).astype(jnp.float32)
        words = jax.lax.dot_general(
            m, gm, (((1,), (0,)), ((), ())),
            preferred_element_type=jnp.float32)
        o_ref[...] = words.astype(jnp.int32)[None]

    return pl.pallas_call(
        body,
        grid=(n_rois,),
        in_specs=[
            pl.BlockSpec((1, 1, 16), lambda n: (n, 0, 0)),
            pl.BlockSpec((nrows, CB), lambda n: (0, 0)),
            pl.BlockSpec((nrows, CB), lambda n: (0, 0)),
            pl.BlockSpec((nrows, CB), lambda n: (0, 0)),
        ],
        out_specs=pl.BlockSpec((1, nrows, W16), lambda n: (n, 0, 0)),
        out_shape=jax.ShapeDtypeStruct((n_rois, nrows, W16), jnp.int32),
    )(params3, x2, y2, z2)


def _pool_sc(params, mwords, ptx, pty, ptz, pts_feature, n_rois):
    P, C = pts_feature.shape
    assert C == 2 * L
    assert n_rois % 32 == 0 and n_rois // 32 == 2
    NW = P // L
    pbits = (P - 1).bit_length()
    mesh = plsc.VectorSubcoreMesh(core_axis_name="c", subcore_axis_name="s")

    @functools.partial(
        pl.kernel,
        out_type=jax.ShapeDtypeStruct((n_rois, VOX * C), jnp.float32),
        mesh=mesh,
        scratch_types=[
            pltpu.VMEM((2 * 16,), jnp.float32),
            pltpu.VMEM((3, P), jnp.float32),
            pltpu.VMEM((2, NW), jnp.int32),
            pltpu.VMEM((2, 2, NW), jnp.int32),
            pltpu.VMEM((2, VOX * C), jnp.float32),
            pltpu.VMEM((2, VOX), jnp.int32),
            pltpu.VMEM((2, P), jnp.int32),
            pltpu.VMEM((L, C), jnp.float32),
            pltpu.VMEM((L,), jnp.int32),
            pltpu.SemaphoreType.DMA,
        ],
        compiler_params=pltpu.CompilerParams(
            needs_layout_passes=False, use_tc_tiling_on_sc=False),
    )
    def body(params_hbm, mask_hbm, ptx_hbm, pty_hbm, ptz_hbm, feat_hbm,
             out_hbm, prm, ptsv, mw, wl, grid, hit, lst, rows, idxb, sem):
        wid = lax.axis_index("s") * 2 + lax.axis_index("c")
        r0 = wid * 2
        with jax.named_scope("stage"):
            pltpu.sync_copy(ptx_hbm, ptsv.at[0])
            pltpu.sync_copy(pty_hbm, ptsv.at[1])
            pltpu.sync_copy(ptz_hbm, ptsv.at[2])
            pltpu.sync_copy(params_hbm.at[pl.ds(r0 * 16, 2 * 16)], prm)
            pltpu.sync_copy(mask_hbm.at[pl.ds(r0, 2)], mw)

        lane = lax.iota(jnp.int32, L)
        zf = jnp.zeros((L,), jnp.float32)
        ziv = jnp.zeros((L,), jnp.int32)

        def zero_grid(i, carry):
            base = i * (4 * L)
            for r in range(2):
                for u in range(4):
                    grid[r, pl.ds(base + u * L, L)] = zf
            return carry

        with jax.named_scope("zero"):
            lax.fori_loop(0, VOX * C // (4 * L), zero_grid, 0)

            def zero_hit(i, carry):
                hit[0, pl.ds(i * L, L)] = ziv
                hit[1, pl.ds(i * L, L)] = ziv
                return carry

            lax.fori_loop(0, VOX // L, zero_hit, 0)

        dn = lax.GatherDimensionNumbers(
            offset_dims=(), collapsed_slice_dims=(0,), start_index_map=(0,))

        def splat(vec, k):
            idx = jnp.full((L, 1), k, jnp.int32)
            return lax.gather(vec, idx, dn, (1,),
                              mode=lax.GatherScatterMode.PROMISE_IN_BOUNDS)

        pv = [prm[pl.ds(0, L)], prm[pl.ds(L, L)]]
        pr = [[splat(pv[r], k) for k in range(11)] for r in range(2)]

        def fullv(v):
            return jnp.full((L,), v, jnp.int32)

        def w_body(g, cws):
            news = []
            for r in range(2):
                wv = mw[r, pl.ds(g * L, L)]
                mk = wv != 0
                mi = mk.astype(jnp.int32)
                incl = plsc.cumsum(mi)
                pos = cws[r] + (incl - mi)
                plsc.store_scatter(wl, [fullv(r), fullv(0), pos],
                                   g * L + lane, mask=mk)
                plsc.store_scatter(wl, [fullv(r), fullv(1), pos],
                                   wv, mask=mk)
                news.append(cws[r] + plsc.all_reduce_population_count(mk))
            return tuple(news)

        with jax.named_scope("wsel"):
            wv_a, wv_b = lax.fori_loop(0, NW // L, w_body, (ziv, ziv))
            nw_a = jnp.max(wv_a)
            nw_b = jnp.max(wv_b)

        def clampi(t, hi):
            ti = t.astype(jnp.int32)
            return jnp.minimum(jnp.maximum(ti, 0), hi)

        def fine_roi(r, nw):
            cx, cy, cz, ca, sa, hdx, hdy, dz, xre, yre, zre = pr[r]

            def f_body(j, cv):
                wsp = plsc.load_gather(wl, [fullv(r), fullv(0), fullv(j)])
                wid_s = jnp.max(wsp)
                wvs = plsc.load_gather(wl, [fullv(r), fullv(1), fullv(j)])
                vm = ((wvs >> lane) & 1) == 1
                base = wid_s * L
                x = ptsv[0, pl.ds(base, L)]
                y = ptsv[1, pl.ds(base, L)]
                z = ptsv[2, pl.ds(base, L)]
                pid = base + lane
                px = x - cx
                py = y - cy
                pz = z - cz
                lx = px * ca - py * sa
                ly = px * sa + py * ca
                xi = clampi((lx + hdx) / xre, OX - 1)
                yi = clampi((ly + hdy) / yre, OY - 1)
                zvi = clampi(pz / zre, OZ - 1)
                vox = (xi * OY + yi) * OZ + zvi
                pk = (vox << pbits) | pid
                mi = vm.astype(jnp.int32)
                incl = plsc.cumsum(mi)
                pos = cv + (incl - mi)
                plsc.store_scatter(lst, [fullv(r), pos], pk, mask=vm)
                return cv + plsc.all_reduce_population_count(vm)

            cv = lax.fori_loop(0, nw, f_body, ziv)
            return jnp.max(cv)

        with jax.named_scope("fine"):
            cnt_a = fine_roi(0, nw_a)
            cnt_b = fine_roi(1, nw_b)

        def apply_roi(r, cnt):
            rs = fullv(r)
            ones = jnp.ones((L,), jnp.int32)

            def ch_body(c, carry):
                base = c * L
                pk = lst[r, pl.ds(base, L)]
                rem = cnt - base
                valid = lane < rem
                idxb[...] = jnp.where(valid, pk & (P - 1), 0)
                pltpu.async_copy(feat_hbm.at[idxb], rows, sem).wait()
                for j in range(L):
                    pkj = plsc.load_gather(lst, [rs, fullv(base + j)])
                    voxj = jnp.minimum(
                        jnp.maximum(pkj >> pbits, 0), VOX - 1)
                    vj = fullv(j) < rem
                    idx0 = voxj * C + lane
                    idx1 = idx0 + L
                    hv = plsc.load_gather(hit, [rs, voxj])
                    g0 = plsc.load_gather(grid, [rs, idx0])
                    g1 = plsc.load_gather(grid, [rs, idx1])
                    f0 = rows[j, pl.ds(0, L)]
                    f1 = rows[j, pl.ds(L, L)]
                    hb = hv > 0
                    n0 = jnp.maximum(jnp.where(hb, g0, NEG), f0)
                    n1 = jnp.maximum(jnp.where(hb, g1, NEG), f1)
                    plsc.store_scatter(grid, [rs, idx0], n0, mask=vj)
                    plsc.store_scatter(grid, [rs, idx1], n1, mask=vj)
                    plsc.store_scatter(hit, [rs, voxj], ones, mask=vj)
                return carry

            nch = (cnt + (L - 1)) >> 4
            lax.fori_loop(0, nch, ch_body, 0)

        with jax.named_scope("p2"):
            apply_roi(0, cnt_a)
            apply_roi(1, cnt_b)

        with jax.named_scope("wout"):
            pltpu.sync_copy(grid.at[0], out_hbm.at[r0])
            pltpu.sync_copy(grid.at[1], out_hbm.at[r0 + 1])

    return body(params, mwords, ptx, pty, ptz, pts_feature)


def kernel(rois, pts, pts_feature):
    n = rois.shape[0]
    p, c = pts_feature.shape
    cx, cy, cz = rois[:, 0], rois[:, 1], rois[:, 2]
    dx, dy, dz = rois[:, 3], rois[:, 4], rois[:, 5]
    rz = rois[:, 6]
    cosa = jnp.cos(-rz)
    sina = jnp.sin(-rz)
    zpad = jnp.zeros_like(cx)
    params = jnp.stack(
        [cx, cy, cz, cosa, sina, dx * 0.5, dy * 0.5, dz,
         dx / OX, dy / OY, dz / OZ,
         zpad, zpad, zpad, zpad, zpad], axis=1).astype(jnp.float32)
    ptsf = pts.astype(jnp.float32)
    xcol, ycol, zcol = ptsf[:, 0], ptsf[:, 1], ptsf[:, 2]
    mwords = _mask_tc(params.reshape(n, 1, 16),
                      xcol.reshape(p // 128, 128),
                      ycol.reshape(p // 128, 128),
                      zcol.reshape(p // 128, 128), n, p)
    pooled = _pool_sc(params.reshape(-1), mwords.reshape(n, p // L),
                      xcol, ycol, zcol,
                      pts_feature.astype(jnp.float32), n)
    return pooled.reshape(n, OX, OY, OZ, c)


# speedup vs baseline: 1.3892x; 1.3892x over previous
import functools

import jax
import jax.numpy as jnp
from jax import lax
from jax.experimental import pallas as pl
from jax.experimental.pallas import tpu as pltpu
from jax.experimental.pallas import tpu_sc as plsc

OX, OY, OZ = 8, 8, 8
VOX = OX * OY * OZ
L = 16
NEG = -3.0e38


def _mask_tc(params, x3, y3, z3, n_rois, P):
    CHUNK = 2048
    WPC = CHUNK // L
    nsteps = P // CHUNK

    def body(prm_ref, x_ref, y_ref, z_ref, o_ref):
        x = x_ref[0]
        y = y_ref[0]
        z = z_ref[0]
        cx = prm_ref[:, 0:1]
        cy = prm_ref[:, 1:2]
        cz = prm_ref[:, 2:3]
        ca = prm_ref[:, 3:4]
        sa = prm_ref[:, 4:5]
        hdx = prm_ref[:, 5:6]
        hdy = prm_ref[:, 6:7]
        dz = prm_ref[:, 7:8]
        px = x - cx
        py = y - cy
        pz = z - cz
        lx = px * ca - py * sa
        ly = px * sa + py * ca
        inb = ((jnp.abs(lx) < hdx) & (jnp.abs(ly) < hdy)
               & (pz > 0.0) & (pz < dz))
        bit = lax.broadcasted_iota(jnp.int32, (n_rois, CHUNK), 1) & (L - 1)
        pwf = (jnp.int32(1) << bit).astype(jnp.float32)
        m = inb.astype(jnp.float32) * pwf
        gi = lax.broadcasted_iota(jnp.int32, (CHUNK, WPC), 0) // L
        gj = lax.broadcasted_iota(jnp.int32, (CHUNK, WPC), 1)
        gm = (gi == gj).astype(jnp.float32)
        words = jax.lax.dot_general(
            m, gm, (((1,), (0,)), ((), ())),
            preferred_element_type=jnp.float32)
        o_ref[...] = words.astype(jnp.int32)

    return pl.pallas_call(
        body,
        grid=(nsteps,),
        in_specs=[
            pl.BlockSpec((n_rois, 16), lambda k: (0, 0)),
            pl.BlockSpec((1, 1, CHUNK), lambda k: (k, 0, 0)),
            pl.BlockSpec((1, 1, CHUNK), lambda k: (k, 0, 0)),
            pl.BlockSpec((1, 1, CHUNK), lambda k: (k, 0, 0)),
        ],
        out_specs=pl.BlockSpec((n_rois, WPC), lambda k: (0, k)),
        out_shape=jax.ShapeDtypeStruct((n_rois, P // L), jnp.int32),
    )(params, x3, y3, z3)


def _pool_sc(params, mwords, ptx, pty, ptz, pts_feature, n_rois):
    P, C = pts_feature.shape
    assert C == 2 * L
    assert n_rois % 32 == 0 and n_rois // 32 == 2
    NW = P // L
    pbits = (P - 1).bit_length()
    mesh = plsc.VectorSubcoreMesh(core_axis_name="c", subcore_axis_name="s")

    @functools.partial(
        pl.kernel,
        out_type=jax.ShapeDtypeStruct((n_rois, VOX * C), jnp.float32),
        mesh=mesh,
        scratch_types=[
            pltpu.VMEM((2 * 16,), jnp.float32),
            pltpu.VMEM((3, P), jnp.float32),
            pltpu.VMEM((2, NW), jnp.int32),
            pltpu.VMEM((2, 2, NW), jnp.int32),
            pltpu.VMEM((2, VOX * C), jnp.float32),
            pltpu.VMEM((2, VOX), jnp.int32),
            pltpu.VMEM((2, P), jnp.int32),
            pltpu.VMEM((L, C), jnp.float32),
            pltpu.VMEM((L,), jnp.int32),
            pltpu.SemaphoreType.DMA,
        ],
        compiler_params=pltpu.CompilerParams(
            needs_layout_passes=False, use_tc_tiling_on_sc=False),
    )
    def body(params_hbm, mask_hbm, ptx_hbm, pty_hbm, ptz_hbm, feat_hbm,
             out_hbm, prm, ptsv, mw, wl, grid, hit, lst, rows, idxb, sem):
        wid = lax.axis_index("s") * 2 + lax.axis_index("c")
        r0 = wid * 2
        with jax.named_scope("stage"):
            pltpu.sync_copy(ptx_hbm, ptsv.at[0])
            pltpu.sync_copy(pty_hbm, ptsv.at[1])
            pltpu.sync_copy(ptz_hbm, ptsv.at[2])
            pltpu.sync_copy(params_hbm.at[pl.ds(r0 * 16, 2 * 16)], prm)
            pltpu.sync_copy(mask_hbm.at[pl.ds(r0, 2)], mw)

        lane = lax.iota(jnp.int32, L)
        zf = jnp.zeros((L,), jnp.float32)
        ziv = jnp.zeros((L,), jnp.int32)

        def zero_grid(i, carry):
            base = i * (4 * L)
            for r in range(2):
                for u in range(4):
                    grid[r, pl.ds(base + u * L, L)] = zf
            return carry

        with jax.named_scope("zero"):
            lax.fori_loop(0, VOX * C // (4 * L), zero_grid, 0)

            def zero_hit(i, carry):
                hit[0, pl.ds(i * L, L)] = ziv
                hit[1, pl.ds(i * L, L)] = ziv
                return carry

            lax.fori_loop(0, VOX // L, zero_hit, 0)

        dn = lax.GatherDimensionNumbers(
            offset_dims=(), collapsed_slice_dims=(0,), start_index_map=(0,))

        def splat(vec, k):
            idx = jnp.full((L, 1), k, jnp.int32)
            return lax.gather(vec, idx, dn, (1,),
                              mode=lax.GatherScatterMode.PROMISE_IN_BOUNDS)

        pv = [prm[pl.ds(0, L)], prm[pl.ds(L, L)]]
        pr = [[splat(pv[r], k) for k in range(11)] for r in range(2)]

        def fullv(v):
            return jnp.full((L,), v, jnp.int32)

        def w_body(g, cws):
            news = []
            for r in range(2):
                wv = mw[r, pl.ds(g * L, L)]
                mk = wv != 0
                mi = mk.astype(jnp.int32)
                incl = plsc.cumsum(mi)
                pos = cws[r] + (incl - mi)
                plsc.store_scatter(wl, [fullv(r), fullv(0), pos],
                                   g * L + lane, mask=mk)
                plsc.store_scatter(wl, [fullv(r), fullv(1), pos],
                                   wv, mask=mk)
                news.append(cws[r] + plsc.all_reduce_population_count(mk))
            return tuple(news)

        with jax.named_scope("wsel"):
            wv_a, wv_b = lax.fori_loop(0, NW // L, w_body, (ziv, ziv))
            nw_a = jnp.max(wv_a)
            nw_b = jnp.max(wv_b)

        def clampi(t, hi):
            ti = t.astype(jnp.int32)
            return jnp.minimum(jnp.maximum(ti, 0), hi)

        def fine_roi(r, nw):
            cx, cy, cz, ca, sa, hdx, hdy, dz, xre, yre, zre = pr[r]

            def f_body(j, cv):
                wsp = plsc.load_gather(wl, [fullv(r), fullv(0), fullv(j)])
                wid_s = jnp.max(wsp)
                wvs = plsc.load_gather(wl, [fullv(r), fullv(1), fullv(j)])
                vm = ((wvs >> lane) & 1) == 1
                base = wid_s * L
                x = ptsv[0, pl.ds(base, L)]
                y = ptsv[1, pl.ds(base, L)]
                z = ptsv[2, pl.ds(base, L)]
                pid = base + lane
                px = x - cx
                py = y - cy
                pz = z - cz
                lx = px * ca - py * sa
                ly = px * sa + py * ca
                xi = clampi((lx + hdx) / xre, OX - 1)
                yi = clampi((ly + hdy) / yre, OY - 1)
                zvi = clampi(pz / zre, OZ - 1)
                vox = (xi * OY + yi) * OZ + zvi
                pk = (vox << pbits) | pid
                mi = vm.astype(jnp.int32)
                incl = plsc.cumsum(mi)
                pos = cv + (incl - mi)
                plsc.store_scatter(lst, [fullv(r), pos], pk, mask=vm)
                return cv + plsc.all_reduce_population_count(vm)

            cv = lax.fori_loop(0, nw, f_body, ziv)
            return jnp.max(cv)

        with jax.named_scope("fine"):
            cnt_a = fine_roi(0, nw_a)
            cnt_b = fine_roi(1, nw_b)

        def apply_roi(r, cnt):
            rs = fullv(r)
            ones = jnp.ones((L,), jnp.int32)

            def ch_body(c, carry):
                base = c * L
                pk = lst[r, pl.ds(base, L)]
                rem = cnt - base
                valid = lane < rem
                idxb[...] = jnp.where(valid, pk & (P - 1), 0)
                pltpu.async_copy(feat_hbm.at[idxb], rows, sem).wait()
                for j in range(L):
                    pkj = plsc.load_gather(lst, [rs, fullv(base + j)])
                    voxj = jnp.minimum(
                        jnp.maximum(pkj >> pbits, 0), VOX - 1)
                    vj = fullv(j) < rem
                    idx0 = voxj * C + lane
                    idx1 = idx0 + L
                    hv = plsc.load_gather(hit, [rs, voxj])
                    g0 = plsc.load_gather(grid, [rs, idx0])
                    g1 = plsc.load_gather(grid, [rs, idx1])
                    f0 = rows[j, pl.ds(0, L)]
                    f1 = rows[j, pl.ds(L, L)]
                    hb = hv > 0
                    n0 = jnp.maximum(jnp.where(hb, g0, NEG), f0)
                    n1 = jnp.maximum(jnp.where(hb, g1, NEG), f1)
                    plsc.store_scatter(grid, [rs, idx0], n0, mask=vj)
                    plsc.store_scatter(grid, [rs, idx1], n1, mask=vj)
                    plsc.store_scatter(hit, [rs, voxj], ones, mask=vj)
                return carry

            nch = (cnt + (L - 1)) >> 4
            lax.fori_loop(0, nch, ch_body, 0)

        with jax.named_scope("p2"):
            apply_roi(0, cnt_a)
            apply_roi(1, cnt_b)

        with jax.named_scope("wout"):
            pltpu.sync_copy(grid.at[0], out_hbm.at[r0])
            pltpu.sync_copy(grid.at[1], out_hbm.at[r0 + 1])

    return body(params, mwords, ptx, pty, ptz, pts_feature)


def kernel(rois, pts, pts_feature):
    n = rois.shape[0]
    p, c = pts_feature.shape
    cx, cy, cz = rois[:, 0], rois[:, 1], rois[:, 2]
    dx, dy, dz = rois[:, 3], rois[:, 4], rois[:, 5]
    rz = rois[:, 6]
    cosa = jnp.cos(-rz)
    sina = jnp.sin(-rz)
    zpad = jnp.zeros_like(cx)
    params = jnp.stack(
        [cx, cy, cz, cosa, sina, dx * 0.5, dy * 0.5, dz,
         dx / OX, dy / OY, dz / OZ,
         zpad, zpad, zpad, zpad, zpad], axis=1).astype(jnp.float32)
    ptsf = pts.astype(jnp.float32)
    xcol, ycol, zcol = ptsf[:, 0], ptsf[:, 1], ptsf[:, 2]
    mwords = _mask_tc(params,
                      xcol.reshape(p // 2048, 1, 2048),
                      ycol.reshape(p // 2048, 1, 2048),
                      zcol.reshape(p // 2048, 1, 2048), n, p)
    pooled = _pool_sc(params.reshape(-1), mwords,
                      xcol, ycol, zcol,
                      pts_feature.astype(jnp.float32), n)
    return pooled.reshape(n, OX, OY, OZ, c)


# speedup vs baseline: 1.4470x; 1.0417x over previous
import functools

import jax
import jax.numpy as jnp
from jax import lax
from jax.experimental import pallas as pl
from jax.experimental.pallas import tpu as pltpu
from jax.experimental.pallas import tpu_sc as plsc

OX, OY, OZ = 8, 8, 8
VOX = OX * OY * OZ
L = 16
NEG = -3.0e38


def _mask_tc(params, x3, y3, z3, n_rois, P):
    CHUNK = 2048
    WPC = CHUNK // L
    nsteps = P // CHUNK

    def body(prm_ref, x_ref, y_ref, z_ref, o_ref):
        x = x_ref[0]
        y = y_ref[0]
        z = z_ref[0]
        cx = prm_ref[:, 0:1]
        cy = prm_ref[:, 1:2]
        cz = prm_ref[:, 2:3]
        ca = prm_ref[:, 3:4]
        sa = prm_ref[:, 4:5]
        hdx = prm_ref[:, 5:6]
        hdy = prm_ref[:, 6:7]
        dz = prm_ref[:, 7:8]
        px = x - cx
        py = y - cy
        pz = z - cz
        lx = px * ca - py * sa
        ly = px * sa + py * ca
        inb = ((jnp.abs(lx) < hdx) & (jnp.abs(ly) < hdy)
               & (pz > 0.0) & (pz < dz))
        bit = lax.broadcasted_iota(jnp.int32, (n_rois, CHUNK), 1) & (L - 1)
        pwf = (jnp.int32(1) << bit).astype(jnp.float32)
        m = inb.astype(jnp.float32) * pwf
        gi = lax.broadcasted_iota(jnp.int32, (CHUNK, WPC), 0) // L
        gj = lax.broadcasted_iota(jnp.int32, (CHUNK, WPC), 1)
        gm = (gi == gj).astype(jnp.float32)
        words = jax.lax.dot_general(
            m, gm, (((1,), (0,)), ((), ())),
            preferred_element_type=jnp.float32)
        o_ref[...] = words.astype(jnp.int32)

    return pl.pallas_call(
        body,
        grid=(nsteps,),
        in_specs=[
            pl.BlockSpec((n_rois, 16), lambda k: (0, 0)),
            pl.BlockSpec((1, 1, CHUNK), lambda k: (k, 0, 0)),
            pl.BlockSpec((1, 1, CHUNK), lambda k: (k, 0, 0)),
            pl.BlockSpec((1, 1, CHUNK), lambda k: (k, 0, 0)),
        ],
        out_specs=pl.BlockSpec((n_rois, WPC), lambda k: (0, k)),
        out_shape=jax.ShapeDtypeStruct((n_rois, P // L), jnp.int32),
    )(params, x3, y3, z3)


def _pool_sc(params, mwords, ptx, pty, ptz, pts_feature, n_rois):
    P, C = pts_feature.shape
    assert C == 2 * L
    assert n_rois % 32 == 0 and n_rois // 32 == 2
    NW = P // L
    pbits = (P - 1).bit_length()
    mesh = plsc.VectorSubcoreMesh(core_axis_name="c", subcore_axis_name="s")

    @functools.partial(
        pl.kernel,
        out_type=jax.ShapeDtypeStruct((n_rois, VOX * C), jnp.float32),
        mesh=mesh,
        scratch_types=[
            pltpu.VMEM((2 * 16,), jnp.float32),
            pltpu.VMEM((3, P), jnp.float32),
            pltpu.VMEM((2, NW), jnp.int32),
            pltpu.VMEM((2, 2, NW), jnp.int32),
            pltpu.VMEM((2, VOX * C), jnp.float32),
            pltpu.VMEM((2, VOX), jnp.int32),
            pltpu.VMEM((2, P), jnp.int32),
            pltpu.VMEM((L, C), jnp.float32),
            pltpu.VMEM((L,), jnp.int32),
            pltpu.SemaphoreType.DMA,
        ],
        compiler_params=pltpu.CompilerParams(
            needs_layout_passes=False, use_tc_tiling_on_sc=False),
    )
    def body(params_hbm, mask_hbm, ptx_hbm, pty_hbm, ptz_hbm, feat_hbm,
             out_hbm, prm, ptsv, mw, wl, grid, hit, lst, rows, idxb, sem):
        wid = lax.axis_index("s") * 2 + lax.axis_index("c")
        r0 = wid * 2
        with jax.named_scope("stage"):
            cpx = pltpu.async_copy(ptx_hbm, ptsv.at[0], sem)
            cpy = pltpu.async_copy(pty_hbm, ptsv.at[1], sem)
            cpz = pltpu.async_copy(ptz_hbm, ptsv.at[2], sem)
            pltpu.sync_copy(params_hbm.at[pl.ds(r0 * 16, 2 * 16)], prm)
            pltpu.sync_copy(mask_hbm.at[pl.ds(r0, 2)], mw)

        lane = lax.iota(jnp.int32, L)
        zf = jnp.zeros((L,), jnp.float32)
        ziv = jnp.zeros((L,), jnp.int32)

        def zero_grid(i, carry):
            base = i * (4 * L)
            for r in range(2):
                for u in range(4):
                    grid[r, pl.ds(base + u * L, L)] = zf
            return carry

        with jax.named_scope("zero"):
            lax.fori_loop(0, VOX * C // (4 * L), zero_grid, 0)

            def zero_hit(i, carry):
                hit[0, pl.ds(i * L, L)] = ziv
                hit[1, pl.ds(i * L, L)] = ziv
                return carry

            lax.fori_loop(0, VOX // L, zero_hit, 0)

        dn = lax.GatherDimensionNumbers(
            offset_dims=(), collapsed_slice_dims=(0,), start_index_map=(0,))

        def splat(vec, k):
            idx = jnp.full((L, 1), k, jnp.int32)
            return lax.gather(vec, idx, dn, (1,),
                              mode=lax.GatherScatterMode.PROMISE_IN_BOUNDS)

        pv = [prm[pl.ds(0, L)], prm[pl.ds(L, L)]]
        pr = [[splat(pv[r], k) for k in range(11)] for r in range(2)]

        def fullv(v):
            return jnp.full((L,), v, jnp.int32)

        def w_body(g, cws):
            news = []
            for r in range(2):
                wv = mw[r, pl.ds(g * L, L)]
                mk = wv != 0
                mi = mk.astype(jnp.int32)
                incl = plsc.cumsum(mi)
                pos = cws[r] + (incl - mi)
                plsc.store_scatter(wl, [fullv(r), fullv(0), pos],
                                   g * L + lane, mask=mk)
                plsc.store_scatter(wl, [fullv(r), fullv(1), pos],
                                   wv, mask=mk)
                news.append(cws[r] + plsc.all_reduce_population_count(mk))
            return tuple(news)

        with jax.named_scope("wsel"):
            wv_a, wv_b = lax.fori_loop(0, NW // L, w_body, (ziv, ziv))
            nw_a = jnp.max(wv_a)
            nw_b = jnp.max(wv_b)

        def clampi(t, hi):
            ti = t.astype(jnp.int32)
            return jnp.minimum(jnp.maximum(ti, 0), hi)

        def fine_roi(r, nw):
            cx, cy, cz, ca, sa, hdx, hdy, dz, xre, yre, zre = pr[r]

            def f_body(j, cv):
                wsp = plsc.load_gather(wl, [fullv(r), fullv(0), fullv(j)])
                wid_s = jnp.max(wsp)
                wvs = plsc.load_gather(wl, [fullv(r), fullv(1), fullv(j)])
                vm = ((wvs >> lane) & 1) == 1
                base = wid_s * L
                x = ptsv[0, pl.ds(base, L)]
                y = ptsv[1, pl.ds(base, L)]
                z = ptsv[2, pl.ds(base, L)]
                pid = base + lane
                px = x - cx
                py = y - cy
                pz = z - cz
                lx = px * ca - py * sa
                ly = px * sa + py * ca
                xi = clampi((lx + hdx) / xre, OX - 1)
                yi = clampi((ly + hdy) / yre, OY - 1)
                zvi = clampi(pz / zre, OZ - 1)
                vox = (xi * OY + yi) * OZ + zvi
                pk = (vox << pbits) | pid
                mi = vm.astype(jnp.int32)
                incl = plsc.cumsum(mi)
                pos = cv + (incl - mi)
                plsc.store_scatter(lst, [fullv(r), pos], pk, mask=vm)
                return cv + plsc.all_reduce_population_count(vm)

            cv = lax.fori_loop(0, nw, f_body, ziv)
            return jnp.max(cv)

        with jax.named_scope("fine"):
            cpx.wait()
            cpy.wait()
            cpz.wait()
            cnt_a = fine_roi(0, nw_a)
            cnt_b = fine_roi(1, nw_b)

        def apply_roi(r, cnt):
            rs = fullv(r)
            ones = jnp.ones((L,), jnp.int32)

            def ch_body(c, carry):
                base = c * L
                pk = lst[r, pl.ds(base, L)]
                rem = cnt - base
                valid = lane < rem
                idxb[...] = jnp.where(valid, pk & (P - 1), 0)
                pltpu.async_copy(feat_hbm.at[idxb], rows, sem).wait()
                for j in range(L):
                    pkj = plsc.load_gather(lst, [rs, fullv(base + j)])
                    voxj = jnp.minimum(
                        jnp.maximum(pkj >> pbits, 0), VOX - 1)
                    vj = fullv(j) < rem
                    idx0 = voxj * C + lane
                    idx1 = idx0 + L
                    hv = plsc.load_gather(hit, [rs, voxj])
                    g0 = plsc.load_gather(grid, [rs, idx0])
                    g1 = plsc.load_gather(grid, [rs, idx1])
                    f0 = rows[j, pl.ds(0, L)]
                    f1 = rows[j, pl.ds(L, L)]
                    hb = hv > 0
                    n0 = jnp.maximum(jnp.where(hb, g0, NEG), f0)
                    n1 = jnp.maximum(jnp.where(hb, g1, NEG), f1)
                    plsc.store_scatter(grid, [rs, idx0], n0, mask=vj)
                    plsc.store_scatter(grid, [rs, idx1], n1, mask=vj)
                    plsc.store_scatter(hit, [rs, voxj], ones, mask=vj)
                return carry

            nch = (cnt + (L - 1)) >> 4
            lax.fori_loop(0, nch, ch_body, 0)

        with jax.named_scope("p2"):
            apply_roi(0, cnt_a)
            apply_roi(1, cnt_b)

        with jax.named_scope("wout"):
            pltpu.sync_copy(grid.at[0], out_hbm.at[r0])
            pltpu.sync_copy(grid.at[1], out_hbm.at[r0 + 1])

    return body(params, mwords, ptx, pty, ptz, pts_feature)


def kernel(rois, pts, pts_feature):
    n = rois.shape[0]
    p, c = pts_feature.shape
    cx, cy, cz = rois[:, 0], rois[:, 1], rois[:, 2]
    dx, dy, dz = rois[:, 3], rois[:, 4], rois[:, 5]
    rz = rois[:, 6]
    cosa = jnp.cos(-rz)
    sina = jnp.sin(-rz)
    zpad = jnp.zeros_like(cx)
    params = jnp.stack(
        [cx, cy, cz, cosa, sina, dx * 0.5, dy * 0.5, dz,
         dx / OX, dy / OY, dz / OZ,
         zpad, zpad, zpad, zpad, zpad], axis=1).astype(jnp.float32)
    ptsf = pts.astype(jnp.float32)
    xcol, ycol, zcol = ptsf[:, 0], ptsf[:, 1], ptsf[:, 2]
    mwords = _mask_tc(params,
                      xcol.reshape(p // 2048, 1, 2048),
                      ycol.reshape(p // 2048, 1, 2048),
                      zcol.reshape(p // 2048, 1, 2048), n, p)
    pooled = _pool_sc(params.reshape(-1), mwords,
                      xcol, ycol, zcol,
                      pts_feature.astype(jnp.float32), n)
    return pooled.reshape(n, OX, OY, OZ, c)
